# baseline (speedup 1.0000x reference)
import functools

import jax
import jax.numpy as jnp
from jax import lax
from jax.experimental import pallas as pl
from jax.experimental.pallas import tpu as pltpu
from jax.experimental.pallas import tpu_sc as plsc

NC = 2
NS = 16
NW = NC * NS
LANES = 16
BBLK = 128
NBUF = 3
NOBUF = 3


def _gather_body(seq, d, dpad, w_hbm, xt_hbm, out_hbm,
                 xv, pairbuf, outblk, sem_g, sem_o):
    wid = lax.axis_index("s") * NC + lax.axis_index("c")
    b0 = wid * BBLK

    pltpu.sync_copy(xt_hbm.at[:, pl.ds(b0, BBLK)], xv)

    def fire_gather(l, b):
        pltpu.async_copy(
            w_hbm.at[xv.at[l, :]], pairbuf.at[b], sem_g.at[b])

    def wait_gather(b):
        pltpu.make_async_copy(
            w_hbm.at[pl.ds(0, BBLK)], pairbuf.at[b], sem_g.at[b]).wait()

    def wait_out(b):
        pltpu.make_async_copy(
            outblk.at[b], out_hbm.at[0, :, pl.ds(0, BBLK)],
            sem_o.at[b]).wait()

    def compact_transpose(b, ob):
        for bg in range(BBLK // LANES):
            rows = jax.lax.iota(jnp.int32, LANES) + bg * LANES
            for dd in range(d):
                col = jnp.full((LANES,), dd, jnp.int32)
                v = plsc.load_gather(pairbuf.at[b], [rows, col])
                outblk[ob, dd, pl.ds(bg * LANES, LANES)] = v

    fire_gather(0, 0)

    @pl.loop(0, seq)
    def _l(l):
        b = lax.rem(l, NBUF)
        ob = lax.rem(l, NOBUF)
        wait_gather(b)

        @pl.when(l + 1 < seq)
        def _():
            fire_gather(l + 1, lax.rem(l + 1, NBUF))

        @pl.when(l - NOBUF >= 0)
        def _():
            wait_out(ob)

        compact_transpose(b, ob)
        pltpu.async_copy(
            outblk.at[ob], out_hbm.at[l, :, pl.ds(b0, BBLK)], sem_o.at[ob])

    for t in range(max(seq - NOBUF, 0), seq):
        wait_out(t % NOBUF)


def _embedding_gather(x, W):
    V, D = W.shape
    B, S = x.shape
    DPAD = 128
    assert B == NW * BBLK

    Wpad = jnp.pad(W, ((0, 0), (0, DPAD - D))).reshape(V, DPAD)
    xt = jnp.transpose(x)

    mesh = plsc.VectorSubcoreMesh(
        core_axis_name="c", subcore_axis_name="s",
        num_cores=NC, num_subcores=NS)

    body = functools.partial(_gather_body, S, D, DPAD)
    out_phys = pl.kernel(
        body,
        out_type=jax.ShapeDtypeStruct((S, D, B), W.dtype),
        mesh=mesh,
        scratch_types=[
            pltpu.VMEM((S, BBLK), jnp.int32),
            pltpu.VMEM((NBUF, BBLK, DPAD), W.dtype),
            pltpu.VMEM((NOBUF, D, BBLK), W.dtype),
            pltpu.SemaphoreType.DMA((NBUF,)),
            pltpu.SemaphoreType.DMA((NOBUF,)),
        ],
        compiler_params=pltpu.CompilerParams(
            use_tc_tiling_on_sc=True, needs_layout_passes=False),
    )(Wpad, xt)
    return jnp.transpose(out_phys, (2, 0, 1))


def kernel(x, target, text_inputs, W):
    return _embedding_gather(x, W)


# speedup vs baseline: 1.3894x; 1.3894x over previous
import functools

import jax
import jax.numpy as jnp
from jax import lax
from jax.experimental import pallas as pl
from jax.experimental.pallas import tpu as pltpu
from jax.experimental.pallas import tpu_sc as plsc

NC = 2
NS = 16
NW = NC * NS
LANES = 16
BBLK = 128
NBUF = 4
DPAD = 128


def _gather_body(seq, d, w_hbm, xt_hbm, out_hbm,
                 xv, idx2, halfb, pairbuf, outblk, sem_g, sem_o):
    wid = lax.axis_index("s") * NC + lax.axis_index("c")
    b0 = wid * BBLK
    n_groups = BBLK // LANES

    pltpu.sync_copy(xt_hbm.at[:, pl.ds(b0, BBLK)], xv)

    def prep(l, b):
        for bg in range(n_groups):
            xvec = xv[l, pl.ds(bg * LANES, LANES)]
            idx2[b, pl.ds(bg * LANES, LANES)] = lax.shift_right_logical(
                xvec, 1)
            halfb[b, pl.ds(bg * LANES, LANES)] = lax.shift_left(
                lax.bitwise_and(xvec, 1), 6)

    def fire_gather(b):
        pltpu.async_copy(w_hbm.at[idx2.at[b]], pairbuf.at[b], sem_g.at[b])

    def wait_gather(b):
        pltpu.make_async_copy(
            w_hbm.at[pl.ds(0, BBLK)], pairbuf.at[b], sem_g.at[b]).wait()

    def wait_out(b):
        pltpu.make_async_copy(
            outblk.at[b], out_hbm.at[0, :, pl.ds(0, BBLK)],
            sem_o.at[b]).wait()

    def compact_transpose(b):
        for bg in range(n_groups):
            rows = lax.iota(jnp.int32, LANES) + bg * LANES
            cols0 = halfb[b, pl.ds(bg * LANES, LANES)]

            @plsc.parallel_loop(0, d, unroll=8)
            def _t(dd):
                v = plsc.load_gather(pairbuf.at[b], [rows, cols0 + dd])
                outblk[b, dd, pl.ds(bg * LANES, LANES)] = v

    prep(0, 0)
    fire_gather(0)

    @pl.loop(0, seq // NBUF)
    def _g(g):
        for b in range(NBUF):
            l = g * NBUF + b
            nb = (b + 1) % NBUF
            wait_gather(b)

            @pl.when(l + 1 < seq)
            def _():
                prep(l + 1, nb)
                fire_gather(nb)

            @pl.when(g > 0)
            def _():
                wait_out(b)

            compact_transpose(b)
            pltpu.async_copy(
                outblk.at[b], out_hbm.at[l, :, pl.ds(b0, BBLK)],
                sem_o.at[b])

    for b in range(NBUF):
        wait_out(b)


def _embedding_gather(x, W):
    V, D = W.shape
    B, S = x.shape
    assert B == NW * BBLK and S % NBUF == 0

    Wp = W.reshape(V // 2, DPAD)
    xt = jnp.transpose(x)

    mesh = plsc.VectorSubcoreMesh(
        core_axis_name="c", subcore_axis_name="s",
        num_cores=NC, num_subcores=NS)

    body = functools.partial(_gather_body, S, D)
    out_phys = pl.kernel(
        body,
        out_type=jax.ShapeDtypeStruct((S, D, B), W.dtype),
        mesh=mesh,
        scratch_types=[
            pltpu.VMEM((S, BBLK), jnp.int32),
            pltpu.VMEM((NBUF, BBLK), jnp.int32),
            pltpu.VMEM((NBUF, BBLK), jnp.int32),
            pltpu.VMEM((NBUF, BBLK, DPAD), W.dtype),
            pltpu.VMEM((NBUF, D, BBLK), W.dtype),
            pltpu.SemaphoreType.DMA((NBUF,)),
            pltpu.SemaphoreType.DMA((NBUF,)),
        ],
        compiler_params=pltpu.CompilerParams(
            use_tc_tiling_on_sc=True, needs_layout_passes=False),
    )(Wp, xt)
    return jnp.transpose(out_phys, (2, 0, 1))


def kernel(x, target, text_inputs, W):
    return _embedding_gather(x, W)
